# Initial kernel scaffold; baseline (speedup 1.0000x reference)
import jax
import jax.numpy as jnp
from jax.experimental import pallas as pl


def kernel(x, edge_index, W):
    raise NotImplementedError("write your pallas kernel here")


# speedup vs baseline: 18.1579x; 18.1579x over previous
import functools

import jax
import jax.numpy as jnp
from jax import lax
from jax.experimental import pallas as pl
from jax.experimental.pallas import tpu as pltpu
from jax.experimental.pallas import tpu_sc as plsc

NC = 2
NS = 16
NW = NC * NS
LB = 128
BN = 1024


def _ceil_to(a, m):
    return (a + m - 1) // m * m


def _deg_kernel(n_pad, chunks):
    nr = n_pad // NS

    @functools.partial(
        pl.kernel,
        out_type=jax.ShapeDtypeStruct((NC, n_pad), jnp.float32),
        mesh=plsc.VectorSubcoreMesh(core_axis_name="c", subcore_axis_name="s"),
        scratch_types=[
            pltpu.VMEM((chunks, LB), jnp.int32),
            pltpu.VMEM((LB,), jnp.float32),
            pltpu.VMEM((nr,), jnp.float32),
            pltpu.VMEM_SHARED((n_pad,), jnp.float32),
        ],
    )
    def deg_kernel(dst_hbm, degp_hbm, idx_v, ones_v, wb_v, deg_sh):
        c = lax.axis_index("c")
        s = lax.axis_index("s")
        wid = c * NS + s
        ones16 = jnp.ones((16,), jnp.float32)
        zeros16 = jnp.zeros((16,), jnp.float32)
        for j in range(LB // 16):
            ones_v[pl.ds(j * 16, 16)] = ones16

        def zbody(i, carry):
            wb_v[pl.ds(i * 16, 16)] = zeros16
            return carry

        lax.fori_loop(0, nr // 16, zbody, 0)
        pltpu.sync_copy(wb_v, deg_sh.at[pl.ds(s * nr, nr)])
        pltpu.sync_copy(dst_hbm.at[wid], idx_v)
        plsc.subcore_barrier()

        def body(j, carry):
            pltpu.sync_copy(ones_v, deg_sh.at[idx_v.at[j]], add=True)
            return carry

        lax.fori_loop(0, chunks, body, 0)
        plsc.subcore_barrier()
        pltpu.sync_copy(deg_sh.at[pl.ds(s * nr, nr)], wb_v)
        pltpu.sync_copy(wb_v, degp_hbm.at[c, pl.ds(s * nr, nr)])

    return deg_kernel


def _agg_kernel(n_pad, chunks, ncol):
    nr = n_pad // NS

    @functools.partial(
        pl.kernel,
        out_type=jax.ShapeDtypeStruct((NC, n_pad, ncol), jnp.float32),
        mesh=plsc.VectorSubcoreMesh(core_axis_name="c", subcore_axis_name="s"),
        scratch_types=[
            pltpu.VMEM((chunks, LB), jnp.int32),
            pltpu.VMEM((chunks, LB), jnp.int32),
            pltpu.VMEM((LB, ncol), jnp.float32),
            pltpu.VMEM_SHARED((n_pad, ncol), jnp.float32),
            pltpu.SemaphoreType.DMA,
        ],
    )
    def agg_kernel(src_hbm, dst_hbm, hs_hbm, accp_hbm,
                   isrc_v, idst_v, rows_v, acc_sh, sem):
        c = lax.axis_index("c")
        s = lax.axis_index("s")
        wid = c * NS + s
        zeros16 = jnp.zeros((16,), jnp.float32)

        def zbody(r, carry):
            for k in range(ncol // 16):
                rows_v[r, pl.ds(k * 16, 16)] = zeros16
            return carry

        lax.fori_loop(0, LB, zbody, 0)
        for k in range(nr // LB):
            pltpu.sync_copy(rows_v, acc_sh.at[pl.ds(s * nr + k * LB, LB)])
        pltpu.sync_copy(src_hbm.at[wid], isrc_v)
        pltpu.sync_copy(dst_hbm.at[wid], idst_v)
        plsc.subcore_barrier()

        def body(j, carry):
            pltpu.async_copy(hs_hbm.at[isrc_v.at[j]], rows_v, sem).wait()
            pltpu.sync_copy(rows_v, acc_sh.at[idst_v.at[j]], add=True)
            return carry

        lax.fori_loop(0, chunks, body, 0)
        plsc.subcore_barrier()
        for k in range(nr // LB):
            pltpu.sync_copy(acc_sh.at[pl.ds(s * nr + k * LB, LB)], rows_v)
            pltpu.sync_copy(rows_v, accp_hbm.at[c, pl.ds(s * nr + k * LB, LB)])

    return agg_kernel


def _matmul_scale(x_pad, W, degp_t, n_pad):
    nfeat = x_pad.shape[1]
    ncol = W.shape[1]

    def body(x_ref, w_ref, degp_ref, hs_ref):
        degs = degp_ref[...]
        deg = degs[:, 0:1] + degs[:, 1:2] + 1.0
        dis = lax.rsqrt(deg)
        h = jnp.dot(x_ref[...], w_ref[...], preferred_element_type=jnp.float32)
        hs_ref[...] = h * dis

    return pl.pallas_call(
        body,
        grid=(n_pad // BN,),
        in_specs=[
            pl.BlockSpec((BN, nfeat), lambda i: (i, 0)),
            pl.BlockSpec((nfeat, ncol), lambda i: (0, 0)),
            pl.BlockSpec((BN, NC), lambda i: (i, 0)),
        ],
        out_specs=pl.BlockSpec((BN, ncol), lambda i: (i, 0)),
        out_shape=jax.ShapeDtypeStruct((n_pad, ncol), jnp.float32),
    )(x_pad, W, degp_t)


def _combine(degp_t, accp, hs, n_pad, ncol):
    def body(degp_ref, accp_ref, hs_ref, out_ref):
        degs = degp_ref[...]
        deg = degs[:, 0:1] + degs[:, 1:2] + 1.0
        dis = lax.rsqrt(deg)
        tot = (accp_ref[0, :, :ncol] + accp_ref[1, :, :ncol]
               + hs_ref[:, :ncol])
        out_ref[...] = tot * dis

    return pl.pallas_call(
        body,
        grid=(n_pad // BN,),
        in_specs=[
            pl.BlockSpec((BN, NC), lambda i: (i, 0)),
            pl.BlockSpec((NC, BN, 128), lambda i: (0, i, 0)),
            pl.BlockSpec((BN, 128), lambda i: (i, 0)),
        ],
        out_specs=pl.BlockSpec((BN, ncol), lambda i: (i, 0)),
        out_shape=jax.ShapeDtypeStruct((n_pad, ncol), jnp.float32),
    )(degp_t, accp, hs)


def kernel(x, edge_index, W):
    n = x.shape[0]
    e = edge_index.shape[1]
    n_pad = _ceil_to(n, BN * 2)
    e_pad = _ceil_to(e, NW * LB)
    chunks = e_pad // (NW * LB)

    ei = edge_index.astype(jnp.int32)
    pad_e = e_pad - e
    trash = jnp.full((pad_e,), n, dtype=jnp.int32)
    src3 = jnp.concatenate([ei[0], trash]).reshape(NW, chunks, LB)
    dst3 = jnp.concatenate([ei[1], trash]).reshape(NW, chunks, LB)
    x_pad = jnp.pad(x, ((0, n_pad - n), (0, 0)))

    ncol = W.shape[1]
    W_pad = jnp.pad(W, ((0, 0), (0, 128 - ncol)))

    degp = _deg_kernel(n_pad, chunks)(dst3)
    degp_t = degp.T
    hs = _matmul_scale(x_pad, W_pad, degp_t, n_pad)
    accp = _agg_kernel(n_pad, chunks, 128)(src3, dst3, hs)
    out = _combine(degp_t, accp, hs, n_pad, ncol)
    return out[:n]
